# Initial kernel scaffold; baseline (speedup 1.0000x reference)
import jax
import jax.numpy as jnp
from jax.experimental import pallas as pl


def kernel(x, edge_index, W, b):
    raise NotImplementedError("write your pallas kernel here")


# speedup vs baseline: 31.4495x; 31.4495x over previous
import functools

import jax
import jax.numpy as jnp
from jax import lax
from jax.experimental import pallas as pl
from jax.experimental.pallas import tpu as pltpu
from jax.experimental.pallas import tpu_sc as plsc

D = 128
N_NODES = 10000
N_EDGES = 320000

NC = 2
NS = 16
NW = NC * NS
EPW = N_EDGES // NW
LANES = 16

ROW_BLOCK = 10000


def _pq_tc_kernel(x_ref, w_ref, b_ref, out_ref):
    xv = x_ref[...]
    s0 = jnp.sum(xv * w_ref[0:1, :], axis=1) + b_ref[0]
    s1 = jnp.sum(xv * w_ref[1:2, :], axis=1)
    out_ref[...] = jnp.concatenate([s0[None, :], s1[None, :]], axis=0)


def _compute_pq(x, w2, b):
    n = x.shape[0]
    return pl.pallas_call(
        _pq_tc_kernel,
        grid=(n // ROW_BLOCK,),
        in_specs=[
            pl.BlockSpec((ROW_BLOCK, D), lambda i: (i, 0)),
            pl.BlockSpec((2, D), lambda i: (0, 0)),
            pl.BlockSpec(memory_space=pltpu.SMEM),
        ],
        out_specs=pl.BlockSpec((2, ROW_BLOCK), lambda i: (0, i)),
        out_shape=jax.ShapeDtypeStruct((2, n), jnp.float32),
    )(x, w2, b)


@functools.partial(
    pl.kernel,
    out_type=jax.ShapeDtypeStruct((N_EDGES,), jnp.float32),
    mesh=plsc.VectorSubcoreMesh(core_axis_name="c", subcore_axis_name="s"),
    compiler_params=pltpu.CompilerParams(needs_layout_passes=False),
    scratch_types=[
        pltpu.VMEM((N_NODES,), jnp.float32),
        pltpu.VMEM((N_NODES,), jnp.float32),
        pltpu.VMEM((EPW,), jnp.int32),
        pltpu.VMEM((EPW,), jnp.int32),
        pltpu.VMEM((EPW,), jnp.float32),
    ],
)
def _sc_edge_kernel(p_hbm, q_hbm, src_hbm, dst_hbm, out_hbm,
                    p_v, q_v, src_v, dst_v, out_v):
    wid = lax.axis_index("s") * NC + lax.axis_index("c")
    base = wid * EPW
    pltpu.sync_copy(p_hbm, p_v)
    pltpu.sync_copy(q_hbm, q_v)
    pltpu.sync_copy(src_hbm.at[pl.ds(base, EPW)], src_v)
    pltpu.sync_copy(dst_hbm.at[pl.ds(base, EPW)], dst_v)

    def body(i, carry):
        off = pl.multiple_of(i * LANES, LANES)
        si = src_v[pl.ds(off, LANES)]
        di = dst_v[pl.ds(off, LANES)]
        vp = plsc.load_gather(p_v, [si])
        vq = plsc.load_gather(q_v, [di])
        out_v[pl.ds(off, LANES)] = vp + vq
        return carry

    lax.fori_loop(0, EPW // LANES, body, 0)
    pltpu.sync_copy(out_v, out_hbm.at[pl.ds(base, EPW)])


def kernel(x, edge_index, W, b):
    w2 = W[:, 0].reshape(2, D)
    pq = _compute_pq(x, w2, b)
    ei = edge_index.astype(jnp.int32)
    out = _sc_edge_kernel(pq[0], pq[1], ei[0], ei[1])
    return out.reshape(N_EDGES, 1)


# speedup vs baseline: 34.3850x; 1.0933x over previous
import functools

import jax
import jax.numpy as jnp
from jax import lax
from jax.experimental import pallas as pl
from jax.experimental.pallas import tpu as pltpu
from jax.experimental.pallas import tpu_sc as plsc

D = 128
N_NODES = 10000
N_EDGES = 320000

NC = 2
NS = 16
NW = NC * NS
EPW = N_EDGES // NW
LANES = 16

ROW_BLOCK = 10000


def _pq_tc_kernel(x_ref, w_ref, b_ref, out_ref):
    xv = x_ref[...]
    s0 = jnp.sum(xv * w_ref[0:1, :], axis=1) + b_ref[0]
    s1 = jnp.sum(xv * w_ref[1:2, :], axis=1)
    out_ref[...] = jnp.concatenate([s0[None, :], s1[None, :]], axis=0)


def _compute_pq(x, w2, b):
    n = x.shape[0]
    return pl.pallas_call(
        _pq_tc_kernel,
        grid=(n // ROW_BLOCK,),
        in_specs=[
            pl.BlockSpec((ROW_BLOCK, D), lambda i: (i, 0)),
            pl.BlockSpec((2, D), lambda i: (0, 0)),
            pl.BlockSpec(memory_space=pltpu.SMEM),
        ],
        out_specs=pl.BlockSpec((2, ROW_BLOCK), lambda i: (0, i)),
        out_shape=jax.ShapeDtypeStruct((2, n), jnp.float32),
    )(x, w2, b)


@functools.partial(
    pl.kernel,
    out_type=jax.ShapeDtypeStruct((N_EDGES,), jnp.float32),
    mesh=plsc.VectorSubcoreMesh(core_axis_name="c", subcore_axis_name="s"),
    compiler_params=pltpu.CompilerParams(needs_layout_passes=False),
    scratch_types=[
        pltpu.VMEM((N_NODES,), jnp.float32),
        pltpu.VMEM((N_NODES,), jnp.float32),
        pltpu.VMEM((EPW,), jnp.int32),
        pltpu.VMEM((EPW,), jnp.int32),
        pltpu.VMEM((EPW,), jnp.float32),
        pltpu.SemaphoreType.DMA,
        pltpu.SemaphoreType.DMA,
        pltpu.SemaphoreType.DMA,
        pltpu.SemaphoreType.DMA,
    ],
)
def _sc_edge_kernel(p_hbm, q_hbm, src_hbm, dst_hbm, out_hbm,
                    p_v, q_v, src_v, dst_v, out_v, sem0, sem1, sem2, sem3):
    wid = lax.axis_index("s") * NC + lax.axis_index("c")
    base = wid * EPW
    c0 = pltpu.async_copy(p_hbm, p_v, sem0)
    c1 = pltpu.async_copy(q_hbm, q_v, sem1)
    c2 = pltpu.async_copy(src_hbm.at[pl.ds(base, EPW)], src_v, sem2)
    c3 = pltpu.async_copy(dst_hbm.at[pl.ds(base, EPW)], dst_v, sem3)
    c2.wait()
    c3.wait()
    c0.wait()
    c1.wait()

    @plsc.parallel_loop(0, EPW, step=LANES, unroll=8)
    def body(off):
        si = src_v[pl.ds(off, LANES)]
        di = dst_v[pl.ds(off, LANES)]
        vp = plsc.load_gather(p_v, [si])
        vq = plsc.load_gather(q_v, [di])
        out_v[pl.ds(off, LANES)] = vp + vq

    pltpu.sync_copy(out_v, out_hbm.at[pl.ds(base, EPW)])


def kernel(x, edge_index, W, b):
    w2 = W[:, 0].reshape(2, D)
    pq = _compute_pq(x, w2, b)
    ei = edge_index.astype(jnp.int32)
    out = _sc_edge_kernel(pq[0], pq[1], ei[0], ei[1])
    return out.reshape(N_EDGES, 1)


# speedup vs baseline: 49.8318x; 1.4492x over previous
import functools

import jax
import jax.numpy as jnp
from jax import lax
from jax.experimental import pallas as pl
from jax.experimental.pallas import tpu as pltpu
from jax.experimental.pallas import tpu_sc as plsc

D = 128
N_NODES = 10000
N_EDGES = 320000

NC = 2
NS = 16
NW = NC * NS
LANES = 16

N_CHUNKS = N_EDGES // 128
CPW = -(-N_CHUNKS // NW)
EPW = CPW * 128


def _pq_tc_kernel(x_ref, w_ref, b_ref, p_ref, q_ref):
    pq = jax.lax.dot_general(
        w_ref[...], x_ref[...], (((1,), (1,)), ((), ())),
        preferred_element_type=jnp.float32,
    )
    p_ref[...] = pq[0, :] + b_ref[0]
    q_ref[...] = pq[1, :]


def _compute_pq(x, w2, b):
    n = x.shape[0]
    return pl.pallas_call(
        _pq_tc_kernel,
        in_specs=[
            pl.BlockSpec((n, D), lambda: (0, 0)),
            pl.BlockSpec((2, D), lambda: (0, 0)),
            pl.BlockSpec(memory_space=pltpu.SMEM),
        ],
        out_specs=[
            pl.BlockSpec((n,), lambda: (0,)),
            pl.BlockSpec((n,), lambda: (0,)),
        ],
        out_shape=[
            jax.ShapeDtypeStruct((n,), jnp.float32),
            jax.ShapeDtypeStruct((n,), jnp.float32),
        ],
    )(x, w2, b)


@functools.partial(
    pl.kernel,
    out_type=jax.ShapeDtypeStruct((N_EDGES,), jnp.float32),
    mesh=plsc.VectorSubcoreMesh(core_axis_name="c", subcore_axis_name="s"),
    compiler_params=pltpu.CompilerParams(needs_layout_passes=False),
    scratch_types=[
        pltpu.VMEM((N_NODES,), jnp.float32),
        pltpu.VMEM((N_NODES,), jnp.float32),
        pltpu.VMEM((CPW, 2, 128), jnp.int32),
        pltpu.VMEM((EPW,), jnp.float32),
        pltpu.SemaphoreType.DMA,
        pltpu.SemaphoreType.DMA,
        pltpu.SemaphoreType.DMA,
    ],
)
def _sc_edge_kernel(p_hbm, q_hbm, idx_hbm, out_hbm,
                    p_v, q_v, idx_v, out_v, sem0, sem1, sem2):
    wid = lax.axis_index("s") * NC + lax.axis_index("c")
    base_c = jnp.minimum(wid * CPW, N_CHUNKS - CPW)
    c0 = pltpu.async_copy(idx_hbm.at[pl.ds(base_c, CPW)], idx_v, sem0)
    c1 = pltpu.async_copy(p_hbm, p_v, sem1)
    c2 = pltpu.async_copy(q_hbm, q_v, sem2)
    c0.wait()
    c1.wait()
    c2.wait()

    @plsc.parallel_loop(0, CPW * 8, 1, unroll=8)
    def body(k):
        c = k >> 3
        off = (k & 7) * LANES
        si = idx_v[c, 0, pl.ds(off, LANES)]
        di = idx_v[c, 1, pl.ds(off, LANES)]
        vp = plsc.load_gather(p_v, [si])
        vq = plsc.load_gather(q_v, [di])
        out_v[pl.ds(k * LANES, LANES)] = vp + vq

    pltpu.sync_copy(out_v, out_hbm.at[pl.ds(base_c * 128, EPW)])


def kernel(x, edge_index, W, b):
    w2 = W[:, 0].reshape(2, D)
    p, q = _compute_pq(x, w2, b)
    ei = edge_index.astype(jnp.int32)
    idx3 = ei.reshape(2, N_CHUNKS, 128).transpose(1, 0, 2)
    out = _sc_edge_kernel(p, q, idx3)
    return out.reshape(N_EDGES, 1)
